# baseline (speedup 1.0000x reference)
import functools
import numpy as np

import jax
import jax.numpy as jnp
from jax import lax
from jax.experimental import pallas as pl
from jax.experimental.pallas import tpu as pltpu
from jax.experimental.pallas import tpu_sc as plsc

N = 256
DIM_H = 16
CUTOFF = 3.6
L = 16
NT = 16

_O_W1 = 0
_O_B1 = _O_W1 + 16 * 7
_O_W2 = _O_B1 + 16
_O_B2 = _O_W2 + 16 * 16
_O_W3 = _O_B2 + 16
_O_B3 = _O_W3 + 32 * 16
_W_ROWS = ((_O_B3 + 32 + 7) // 8) * 8
_O_WE = 0
_O_BE = _O_WE + 16 * 16
_O_WD = _O_BE + 16
_O_BD = _O_WD + 7 * 16
_D_ROWS = ((_O_BD + 7 + 7) // 8) * 8


def _atan(x):
    t = jnp.abs(x)
    c1 = t > 2.414213562373095
    c2 = t > 0.4142135623730951
    base = jnp.where(c1, np.float32(np.pi / 2),
                     jnp.where(c2, np.float32(np.pi / 4), np.float32(0.0)))
    arg = jnp.where(c1, -1.0 / t, jnp.where(c2, (t - 1.0) / (t + 1.0), t))
    z = arg * arg
    p = (((8.05374449538e-2 * z - 1.38776856032e-1) * z
          + 1.99777106478e-1) * z - 3.33329491539e-1) * z * arg + arg
    return jnp.sign(x) * (base + p)


def _sc_body(w_hbm, wd_hbm, ps_hbm, xt_hbm, out_hbm,
             wv, wdv, posv, xv, tloc, rowsv, fvr, fallv, fmmv, btv, bsv, bbv, av, arow, ov,
             sem1, sem2, sem3, spA, spW, spF, spF2, spB):
    t = lax.axis_index("s")
    f32 = jnp.float32
    iot = lax.broadcasted_iota(jnp.int32, (L,), 0)

    cp_pos = pltpu.async_copy(ps_hbm, posv, sem1)
    cp_dec = pltpu.async_copy(wd_hbm, wdv, sem2)
    cp_w = pltpu.async_copy(w_hbm, wv, sem3)
    pltpu.sync_copy(xt_hbm.at[t], xv)

    zero = jnp.zeros((L,), f32)
    for r in range(2 * L):
        tloc[r] = zero
    xr = [xv[i] for i in range(7)]
    cp_w.wait()

    h1 = []
    for j in range(16):
        acc = wv[_O_B1 + j]
        for i in range(7):
            acc = acc + xr[i] * wv[_O_W1 + j * 7 + i]
        h1.append(_atan(acc))
    h2 = []
    for j in range(16):
        acc = wv[_O_B2 + j]
        for i in range(16):
            acc = acc + h1[i] * wv[_O_W2 + j * 16 + i]
        h2.append(_atan(acc))
    m = []
    for j in range(32):
        acc = wv[_O_B3 + j]
        for i in range(16):
            acc = acc + h2[i] * wv[_O_W3 + j * 16 + i]
        m.append(acc)
    a = m[16] * m[24]
    for j in range(1, 8):
        a = a + m[16 + j] * m[24 + j]

    s = a[0]
    for i in range(1, 16):
        s = jnp.maximum(s, a[i])
    lm = jnp.zeros((L,), f32) + s
    ep = jnp.exp(a - lm)
    arow[0] = a * np.float32(0.0) + lm
    pltpu.sync_copy(arow, spA.at[pl.ds(t, 1)])

    cp_pos.wait()
    big = np.int32(1 << 20)

    def fstep(k4, f):
        for q in range(8):
            kk = 8 * k4 + q
            d = (jnp.abs(posv[3 * kk] - xr[0])
                 + jnp.abs(posv[3 * kk + 1] - xr[1])
                 + jnp.abs(posv[3 * kk + 2] - xr[2]))
            hit = d <= np.float32(CUTOFF)
            f = jnp.minimum(f, jnp.where(hit, kk, big))
        return f

    f = lax.fori_loop(0, N // 8, fstep, jnp.full((L,), 1 << 20, jnp.int32))

    for n in range(16):
        en = ep[n]
        mrow = jnp.where(iot == 0, m[0][n] * en, np.float32(0.0))
        for j in range(1, 16):
            mrow = jnp.where(iot == j, m[j][n] * en, mrow)
        rowsv[2 * n] = mrow
        rowsv[2 * n + 1] = jnp.where(
            iot == 0, en, jnp.where(iot == 1, np.float32(1.0), np.float32(0.0)))
    pltpu.sync_copy(rowsv, spW.at[pl.ds(2 * L * t, 2 * L)])
    fvr[0] = f
    pltpu.sync_copy(fvr, spF.at[pl.ds(t, 1)])
    fmn = f[0]
    fmx = f[0]
    for n in range(1, 16):
        fn = f[n]
        fmn = jnp.minimum(fmn, fn)
        fmx = jnp.maximum(fmx, fn)
    fvr[0] = f * 0 + jnp.where(iot == 0, fmn, fmx)
    pltpu.sync_copy(fvr, spF2.at[pl.ds(t, 1)])

    plsc.subcore_barrier()

    pltpu.sync_copy(spA, av)
    mxv = av[0]
    for c in range(1, NT):
        mxv = jnp.maximum(mxv, av[c])
    amax = jnp.maximum(mxv, np.float32(0.0))
    expneg = jnp.exp(-amax)
    one = np.float32(1.0)
    pltpu.sync_copy(spF, fallv)
    pltpu.sync_copy(spF2, fmmv)
    base = t * L
    for u in range(NT):
        mn_u = fmmv[u][0]
        mx_u = fmmv[u][1]
        hit_u = jnp.logical_and(mn_u < base + L, mx_u >= base)

        @pl.when(hit_u)
        def _(u=u):
            fc = fallv[u]
            rel = fc - base
            sc_u = jnp.exp(av[u] - amax)
            sc_odd = jnp.where(iot == 1, one, sc_u)
            pltpu.sync_copy(spW.at[pl.ds(2 * L * u, 2 * L)], btv)
            for n2 in range(16):
                rn = rel[n2]
                ok = jnp.logical_and(rn >= 0, rn < L)

                @pl.when(ok)
                def _(n2=n2, rn=rn, sc_u=sc_u, sc_odd=sc_odd):
                    tloc[2 * rn] = tloc[2 * rn] + sc_u * btv[2 * n2]
                    tloc[2 * rn + 1] = (tloc[2 * rn + 1]
                                        + sc_odd * btv[2 * n2 + 1])

    cb = [tloc[r] for r in range(2 * L)]
    bs0 = cb[0]
    bs1 = cb[1]
    for r in range(1, 16):
        bs0 = bs0 + cb[2 * r]
        bs1 = bs1 + cb[2 * r + 1]
    bsv[0] = bs0
    bsv[1] = bs1
    pltpu.sync_copy(bsv, spB.at[pl.ds(2 * t, 2)])

    plsc.subcore_barrier()

    pltpu.sync_copy(spB, bbv)
    acc0 = jnp.zeros((L,), f32)
    acc1 = jnp.zeros((L,), f32)
    for u in range(NT):
        g = jnp.where(u < t, np.float32(1.0), np.float32(0.0))
        acc0 = acc0 + g * bbv[2 * u]
        acc1 = acc1 + g * bbv[2 * u + 1]
    srows0 = []
    srows1 = []
    for r in range(16):
        acc0 = acc0 + cb[2 * r]
        acc1 = acc1 + cb[2 * r + 1]
        srows0.append(acc0)
        srows1.append(acc1)

    chans = []
    for j in range(16):
        v = jnp.where(iot == 0, srows0[0][j], np.float32(0.0))
        for r in range(1, 16):
            v = jnp.where(iot == r, srows0[r][j], v)
        chans.append(v)
    pe = jnp.where(iot == 0, srows1[0][0], np.float32(0.0))
    cnt = jnp.where(iot == 0, srows1[0][1], np.float32(0.0))
    for r in range(1, 16):
        pe = jnp.where(iot == r, srows1[r][0], pe)
        cnt = jnp.where(iot == r, srows1[r][1], cnt)

    den = pe + (np.float32(N) - cnt) * expneg
    inv = np.float32(1.0) / den
    my = [chans[j] * inv for j in range(16)]
    cp_dec.wait()
    code = []
    for j in range(16):
        acc = wdv[_O_BE + j]
        for i in range(16):
            acc = acc + my[i] * wdv[_O_WE + j * 16 + i]
        code.append(_atan(acc))
    for r in range(7):
        acc = wdv[_O_BD + r]
        for j in range(16):
            acc = acc + code[j] * wdv[_O_WD + r * 16 + j]
        ov[r] = acc
    ov[7] = zero
    pltpu.sync_copy(ov, out_hbm.at[t])


@jax.jit
def kernel(x, W1, b1, W2, b2, W3, b3, We, be, Wd, bd):
    flat = jnp.concatenate([
        W1.ravel(), b1, W2.ravel(), b2, W3.ravel(), b3,
        jnp.zeros((_W_ROWS - (_O_B3 + 32),), jnp.float32),
    ])
    flatd = jnp.concatenate([
        We.ravel(), be, Wd.ravel(), bd,
        jnp.zeros((_D_ROWS - (_O_BD + 7),), jnp.float32),
    ])
    wsplat = jnp.repeat(flat[:, None], L, axis=1)
    wdsplat = jnp.repeat(flatd[:, None], L, axis=1)
    ps = jnp.repeat(x[:, 0:3].reshape(-1)[:, None], L, axis=1)
    xt = jnp.zeros((NT, 8, L), jnp.float32).at[:, 0:7, :].set(
        x.T.reshape(7, NT, L).transpose(1, 0, 2))

    mesh = plsc.VectorSubcoreMesh(core_axis_name="c", subcore_axis_name="s",
                                  num_cores=1, num_subcores=NT)
    sc = pl.kernel(
        _sc_body,
        out_type=jax.ShapeDtypeStruct((NT, 8, L), jnp.float32),
        mesh=mesh,
        compiler_params=pltpu.CompilerParams(use_tc_tiling_on_sc=False),
        scratch_types=[
            pltpu.VMEM((_W_ROWS, L), jnp.float32),
            pltpu.VMEM((_D_ROWS, L), jnp.float32),
            pltpu.VMEM((3 * N, L), jnp.float32),
            pltpu.VMEM((8, L), jnp.float32),
            pltpu.VMEM((2 * L, L), jnp.float32),
            pltpu.VMEM((2 * L, L), jnp.float32),
            pltpu.VMEM((1, L), jnp.int32),
            pltpu.VMEM((NT, L), jnp.int32),
            pltpu.VMEM((NT, L), jnp.int32),
            pltpu.VMEM((2 * L, L), jnp.float32),
            pltpu.VMEM((2, L), jnp.float32),
            pltpu.VMEM((2 * NT, L), jnp.float32),
            pltpu.VMEM((NT, L), jnp.float32),
            pltpu.VMEM((1, L), jnp.float32),
            pltpu.VMEM((8, L), jnp.float32),
            pltpu.SemaphoreType.DMA,
            pltpu.SemaphoreType.DMA,
            pltpu.SemaphoreType.DMA,
            pltpu.VMEM_SHARED((NT, L), jnp.float32),
            pltpu.VMEM_SHARED((2 * N, L), jnp.float32),
            pltpu.VMEM_SHARED((NT, L), jnp.int32),
            pltpu.VMEM_SHARED((NT, L), jnp.int32),
            pltpu.VMEM_SHARED((2 * NT, L), jnp.float32),
        ],
    )
    out = sc(wsplat, wdsplat, ps, xt)
    return jnp.transpose(out, (0, 2, 1)).reshape(N, 8)[:, :7]


# speedup vs baseline: 1.2118x; 1.2118x over previous
import functools
import numpy as np

import jax
import jax.numpy as jnp
from jax import lax
from jax.experimental import pallas as pl
from jax.experimental.pallas import tpu as pltpu
from jax.experimental.pallas import tpu_sc as plsc

N = 256
DIM_H = 16
CUTOFF = 3.6
L = 16
NT = 16

_O_W1 = 0
_O_B1 = _O_W1 + 16 * 7
_O_W2 = _O_B1 + 16
_O_B2 = _O_W2 + 16 * 16
_O_W3 = _O_B2 + 16
_O_B3 = _O_W3 + 32 * 16
_W_ROWS = ((_O_B3 + 32 + 7) // 8) * 8
_O_WE = 0
_O_BE = _O_WE + 16 * 16
_O_WD = _O_BE + 16
_O_BD = _O_WD + 7 * 16
_D_ROWS = ((_O_BD + 7 + 7) // 8) * 8


def _atan(x):
    t = jnp.abs(x)
    c1 = t > 2.414213562373095
    c2 = t > 0.4142135623730951
    base = jnp.where(c1, np.float32(np.pi / 2),
                     jnp.where(c2, np.float32(np.pi / 4), np.float32(0.0)))
    arg = jnp.where(c1, -1.0 / t, jnp.where(c2, (t - 1.0) / (t + 1.0), t))
    z = arg * arg
    p = (((8.05374449538e-2 * z - 1.38776856032e-1) * z
          + 1.99777106478e-1) * z - 3.33329491539e-1) * z * arg + arg
    return jnp.sign(x) * (base + p)


def _sc_body(w_hbm, wd_hbm, ps_hbm, xt_hbm, out_hbm,
             wv, wdv, posv, xv, tloc, hv, btv, bsv, bbv, av, arow, ov,
             sem1, sem2, sem3, spA, spTT, spB):
    t = lax.axis_index("s")
    f32 = jnp.float32
    iot = lax.broadcasted_iota(jnp.int32, (L,), 0)

    cp_pos = pltpu.async_copy(ps_hbm, posv, sem1)
    cp_dec = pltpu.async_copy(wd_hbm, wdv, sem2)
    cp_w = pltpu.async_copy(w_hbm, wv, sem3)
    pltpu.sync_copy(xt_hbm.at[t], xv)

    zero = jnp.zeros((L,), f32)

    def zstep(k, c):
        for r in range(16):
            tloc[16 * k + r] = zero
        return c

    lax.fori_loop(0, 32, zstep, jnp.int32(0))
    xr = [xv[i] for i in range(7)]
    cp_w.wait()

    def l1(j, c):
        acc = wv[_O_B1 + j]
        for i in range(7):
            acc = acc + xr[i] * wv[_O_W1 + j * 7 + i]
        hv[j] = _atan(acc)
        return c

    lax.fori_loop(0, 16, l1, jnp.int32(0))

    def l2(j, c):
        acc = wv[_O_B2 + j]
        for i in range(16):
            acc = acc + hv[i] * wv[_O_W2 + j * 16 + i]
        hv[16 + j] = _atan(acc)
        return c

    lax.fori_loop(0, 16, l2, jnp.int32(0))

    def l3(j, c):
        acc = wv[_O_B3 + j]
        for i in range(16):
            acc = acc + hv[16 + i] * wv[_O_W3 + j * 16 + i]
        hv[32 + j] = acc
        return c

    lax.fori_loop(0, 32, l3, jnp.int32(0))
    m = [hv[32 + j] for j in range(32)]
    a = m[16] * m[24]
    for j in range(1, 8):
        a = a + m[16 + j] * m[24 + j]

    s = a[0]
    for i in range(1, 16):
        s = jnp.maximum(s, a[i])
    lm = jnp.zeros((L,), f32) + s
    ep = jnp.exp(a - lm)
    arow[0] = a * np.float32(0.0) + lm
    pltpu.sync_copy(arow, spA.at[pl.ds(t, 1)])

    cp_pos.wait()
    big = np.int32(1 << 20)

    def fstep(k4, f):
        for q in range(8):
            kk = 8 * k4 + q
            d = (jnp.abs(posv[3 * kk] - xr[0])
                 + jnp.abs(posv[3 * kk + 1] - xr[1])
                 + jnp.abs(posv[3 * kk + 2] - xr[2]))
            hit = d <= np.float32(CUTOFF)
            f = jnp.minimum(f, jnp.where(hit, kk, big))
        return f

    f = lax.fori_loop(0, N // 8, fstep, jnp.full((L,), 1 << 20, jnp.int32))

    for n in range(16):
        en = ep[n]
        mrow = jnp.where(iot == 0, m[0][n] * en, np.float32(0.0))
        for j in range(1, 16):
            mrow = jnp.where(iot == j, m[j][n] * en, mrow)
        tail = jnp.where(
            iot == 0, en, jnp.where(iot == 1, np.float32(1.0), np.float32(0.0)))
        fn = f[n]
        tloc[2 * fn] = tloc[2 * fn] + mrow
        tloc[2 * fn + 1] = tloc[2 * fn + 1] + tail
    pltpu.sync_copy(tloc, spTT.at[t])

    plsc.subcore_barrier()

    pltpu.sync_copy(spA, av)
    mxv = av[0]
    for c in range(1, NT):
        mxv = jnp.maximum(mxv, av[c])
    amax = jnp.maximum(mxv, np.float32(0.0))
    expneg = jnp.exp(-amax)
    one = np.float32(1.0)
    cb = [jnp.zeros((L,), f32) for _ in range(2 * L)]
    for u in range(NT):
        sc_u = jnp.exp(av[u] - amax)
        sc_odd = jnp.where(iot == 1, one, sc_u)
        pltpu.sync_copy(spTT.at[u, pl.ds(2 * L * t, 2 * L)], btv)
        for r in range(16):
            cb[2 * r] = cb[2 * r] + sc_u * btv[2 * r]
            cb[2 * r + 1] = cb[2 * r + 1] + sc_odd * btv[2 * r + 1]
    bs0 = cb[0]
    bs1 = cb[1]
    for r in range(1, 16):
        bs0 = bs0 + cb[2 * r]
        bs1 = bs1 + cb[2 * r + 1]
    bsv[0] = bs0
    bsv[1] = bs1
    pltpu.sync_copy(bsv, spB.at[pl.ds(2 * t, 2)])

    plsc.subcore_barrier()

    pltpu.sync_copy(spB, bbv)
    acc0 = jnp.zeros((L,), f32)
    acc1 = jnp.zeros((L,), f32)
    for u in range(NT):
        g = jnp.where(u < t, np.float32(1.0), np.float32(0.0))
        acc0 = acc0 + g * bbv[2 * u]
        acc1 = acc1 + g * bbv[2 * u + 1]
    srows0 = []
    srows1 = []
    for r in range(16):
        acc0 = acc0 + cb[2 * r]
        acc1 = acc1 + cb[2 * r + 1]
        srows0.append(acc0)
        srows1.append(acc1)

    chans = []
    for j in range(16):
        v = jnp.where(iot == 0, srows0[0][j], np.float32(0.0))
        for r in range(1, 16):
            v = jnp.where(iot == r, srows0[r][j], v)
        chans.append(v)
    pe = jnp.where(iot == 0, srows1[0][0], np.float32(0.0))
    cnt = jnp.where(iot == 0, srows1[0][1], np.float32(0.0))
    for r in range(1, 16):
        pe = jnp.where(iot == r, srows1[r][0], pe)
        cnt = jnp.where(iot == r, srows1[r][1], cnt)

    den = pe + (np.float32(N) - cnt) * expneg
    inv = np.float32(1.0) / den
    my = [chans[j] * inv for j in range(16)]
    cp_dec.wait()
    for i in range(16):
        hv[i] = my[i]

    def dec1(j, c):
        acc = wdv[_O_BE + j]
        for i in range(16):
            acc = acc + hv[i] * wdv[_O_WE + j * 16 + i]
        hv[16 + j] = _atan(acc)
        return c

    lax.fori_loop(0, 16, dec1, jnp.int32(0))

    def dec2(r, c):
        acc = wdv[_O_BD + r]
        for j in range(16):
            acc = acc + hv[16 + j] * wdv[_O_WD + r * 16 + j]
        ov[r] = acc
        return c

    lax.fori_loop(0, 7, dec2, jnp.int32(0))
    ov[7] = zero
    pltpu.sync_copy(ov, out_hbm.at[t])


@jax.jit
def kernel(x, W1, b1, W2, b2, W3, b3, We, be, Wd, bd):
    flat = jnp.concatenate([
        W1.ravel(), b1, W2.ravel(), b2, W3.ravel(), b3,
        jnp.zeros((_W_ROWS - (_O_B3 + 32),), jnp.float32),
    ])
    flatd = jnp.concatenate([
        We.ravel(), be, Wd.ravel(), bd,
        jnp.zeros((_D_ROWS - (_O_BD + 7),), jnp.float32),
    ])
    wsplat = jnp.repeat(flat[:, None], L, axis=1)
    wdsplat = jnp.repeat(flatd[:, None], L, axis=1)
    ps = jnp.repeat(x[:, 0:3].reshape(-1)[:, None], L, axis=1)
    xt = jnp.zeros((NT, 8, L), jnp.float32).at[:, 0:7, :].set(
        x.T.reshape(7, NT, L).transpose(1, 0, 2))

    mesh = plsc.VectorSubcoreMesh(core_axis_name="c", subcore_axis_name="s",
                                  num_cores=1, num_subcores=NT)
    sc = pl.kernel(
        _sc_body,
        out_type=jax.ShapeDtypeStruct((NT, 8, L), jnp.float32),
        mesh=mesh,
        compiler_params=pltpu.CompilerParams(use_tc_tiling_on_sc=False),
        scratch_types=[
            pltpu.VMEM((_W_ROWS, L), jnp.float32),
            pltpu.VMEM((_D_ROWS, L), jnp.float32),
            pltpu.VMEM((3 * N, L), jnp.float32),
            pltpu.VMEM((8, L), jnp.float32),
            pltpu.VMEM((2 * N, L), jnp.float32),
            pltpu.VMEM((64, L), jnp.float32),
            pltpu.VMEM((2 * L, L), jnp.float32),
            pltpu.VMEM((2, L), jnp.float32),
            pltpu.VMEM((2 * NT, L), jnp.float32),
            pltpu.VMEM((NT, L), jnp.float32),
            pltpu.VMEM((1, L), jnp.float32),
            pltpu.VMEM((8, L), jnp.float32),
            pltpu.SemaphoreType.DMA,
            pltpu.SemaphoreType.DMA,
            pltpu.SemaphoreType.DMA,
            pltpu.VMEM_SHARED((NT, L), jnp.float32),
            pltpu.VMEM_SHARED((NT, 2 * N, L), jnp.float32),
            pltpu.VMEM_SHARED((2 * NT, L), jnp.float32),
        ],
    )
    out = sc(wsplat, wdsplat, ps, xt)
    return jnp.transpose(out, (0, 2, 1)).reshape(N, 8)[:, :7]


# speedup vs baseline: 1.2124x; 1.0005x over previous
import numpy as np

import jax
import jax.numpy as jnp
from jax import lax
from jax.experimental import pallas as pl
from jax.experimental.pallas import tpu as pltpu
from jax.experimental.pallas import tpu_sc as plsc

N = 256
DIM_H = 16
CUTOFF = 3.6
L = 16
NT = 16

_O_W1 = 0
_O_B1 = _O_W1 + 16 * 7
_O_W2 = _O_B1 + 16
_O_B2 = _O_W2 + 16 * 16
_O_W3 = _O_B2 + 16
_O_B3 = _O_W3 + 32 * 16
_W_ROWS = ((_O_B3 + 32 + 7) // 8) * 8
_O_WE = 0
_O_BE = _O_WE + 16 * 16
_O_WD = _O_BE + 16
_O_BD = _O_WD + 7 * 16
_D_ROWS = ((_O_BD + 7 + 7) // 8) * 8


def _atan(x):
    t = jnp.abs(x)
    c1 = t > 2.414213562373095
    c2 = t > 0.4142135623730951
    base = jnp.where(c1, np.float32(np.pi / 2),
                     jnp.where(c2, np.float32(np.pi / 4), np.float32(0.0)))
    arg = jnp.where(c1, -1.0 / t, jnp.where(c2, (t - 1.0) / (t + 1.0), t))
    z = arg * arg
    p = (((8.05374449538e-2 * z - 1.38776856032e-1) * z
          + 1.99777106478e-1) * z - 3.33329491539e-1) * z * arg + arg
    return jnp.sign(x) * (base + p)


def _sc_body(w_hbm, wd_hbm, ps_hbm, xt_hbm, out_hbm,
             wv, wdv, posv, xv, tloc, hv, btv, bsv, bbv, av, arow, ov,
             sem1, sem2, sem3, spA, spTT, spB):
    t = lax.axis_index("s")
    f32 = jnp.float32
    iot = lax.broadcasted_iota(jnp.int32, (L,), 0)

    cp_pos = pltpu.async_copy(ps_hbm, posv, sem1)
    cp_dec = pltpu.async_copy(wd_hbm, wdv, sem2)
    cp_w = pltpu.async_copy(w_hbm, wv, sem3)
    pltpu.sync_copy(xt_hbm.at[t], xv)

    zero = jnp.zeros((L,), f32)

    def zstep(k, c):
        for r in range(16):
            tloc[16 * k + r] = zero
        return c

    lax.fori_loop(0, 32, zstep, jnp.int32(0))
    xr = [xv[i] for i in range(7)]
    cp_w.wait()

    def l1(j, c):
        acc = wv[_O_B1 + j]
        for i in range(7):
            acc = acc + xr[i] * wv[_O_W1 + j * 7 + i]
        hv[j] = _atan(acc)
        return c

    lax.fori_loop(0, 16, l1, jnp.int32(0))

    def l2(j, c):
        acc = wv[_O_B2 + j]
        for i in range(16):
            acc = acc + hv[i] * wv[_O_W2 + j * 16 + i]
        hv[16 + j] = _atan(acc)
        return c

    lax.fori_loop(0, 16, l2, jnp.int32(0))

    def l3(j, c):
        acc = wv[_O_B3 + j]
        for i in range(16):
            acc = acc + hv[16 + i] * wv[_O_W3 + j * 16 + i]
        hv[32 + j] = acc
        return c

    lax.fori_loop(0, 32, l3, jnp.int32(0))
    m = [hv[32 + j] for j in range(32)]
    a = m[16] * m[24]
    for j in range(1, 8):
        a = a + m[16 + j] * m[24 + j]

    s = a[0]
    for i in range(1, 16):
        s = jnp.maximum(s, a[i])
    lm = jnp.zeros((L,), f32) + s
    ep = jnp.exp(a - lm)
    arow[0] = a * np.float32(0.0) + lm
    pltpu.sync_copy(arow, spA.at[pl.ds(t, 1)])

    cp_pos.wait()
    big = np.int32(1 << 20)

    def fstep(k4, f):
        for q in range(8):
            kk = 8 * k4 + q
            d = (jnp.abs(posv[3 * kk] - xr[0])
                 + jnp.abs(posv[3 * kk + 1] - xr[1])
                 + jnp.abs(posv[3 * kk + 2] - xr[2]))
            hit = d <= np.float32(CUTOFF)
            f = jnp.minimum(f, jnp.where(hit, kk, big))
        return f

    f = lax.fori_loop(0, N // 8, fstep, jnp.full((L,), 1 << 20, jnp.int32))

    for n in range(16):
        en = ep[n]
        mrow = jnp.where(iot == 0, m[0][n] * en, np.float32(0.0))
        for j in range(1, 16):
            mrow = jnp.where(iot == j, m[j][n] * en, mrow)
        tail = jnp.where(
            iot == 0, en, jnp.where(iot == 1, np.float32(1.0), np.float32(0.0)))
        fn = f[n]
        tloc[2 * fn] = tloc[2 * fn] + mrow
        tloc[2 * fn + 1] = tloc[2 * fn + 1] + tail
    pltpu.sync_copy(tloc, spTT.at[t])

    plsc.subcore_barrier()

    pltpu.sync_copy(spA, av)
    mxv = av[0]
    for c in range(1, NT):
        mxv = jnp.maximum(mxv, av[c])
    amax = jnp.maximum(mxv, np.float32(0.0))
    expneg = jnp.exp(-amax)
    one = np.float32(1.0)
    cb = [jnp.zeros((L,), f32) for _ in range(2 * L)]
    for u in range(NT):
        sc_u = jnp.exp(av[u] - amax)
        sc_odd = jnp.where(iot == 1, one, sc_u)
        pltpu.sync_copy(spTT.at[u, pl.ds(2 * L * t, 2 * L)], btv)
        for r in range(16):
            cb[2 * r] = cb[2 * r] + sc_u * btv[2 * r]
            cb[2 * r + 1] = cb[2 * r + 1] + sc_odd * btv[2 * r + 1]
    bs0 = cb[0]
    bs1 = cb[1]
    for r in range(1, 16):
        bs0 = bs0 + cb[2 * r]
        bs1 = bs1 + cb[2 * r + 1]
    bsv[0] = bs0
    bsv[1] = bs1
    pltpu.sync_copy(bsv, spB.at[pl.ds(2 * t, 2)])

    plsc.subcore_barrier()

    pltpu.sync_copy(spB, bbv)
    acc0 = jnp.zeros((L,), f32)
    acc1 = jnp.zeros((L,), f32)
    for u in range(NT):
        g = jnp.where(u < t, np.float32(1.0), np.float32(0.0))
        acc0 = acc0 + g * bbv[2 * u]
        acc1 = acc1 + g * bbv[2 * u + 1]
    srows0 = []
    srows1 = []
    for r in range(16):
        acc0 = acc0 + cb[2 * r]
        acc1 = acc1 + cb[2 * r + 1]
        srows0.append(acc0)
        srows1.append(acc1)

    chans = []
    for j in range(16):
        v = jnp.where(iot == 0, srows0[0][j], np.float32(0.0))
        for r in range(1, 16):
            v = jnp.where(iot == r, srows0[r][j], v)
        chans.append(v)
    pe = jnp.where(iot == 0, srows1[0][0], np.float32(0.0))
    cnt = jnp.where(iot == 0, srows1[0][1], np.float32(0.0))
    for r in range(1, 16):
        pe = jnp.where(iot == r, srows1[r][0], pe)
        cnt = jnp.where(iot == r, srows1[r][1], cnt)

    den = pe + (np.float32(N) - cnt) * expneg
    inv = np.float32(1.0) / den
    my = [chans[j] * inv for j in range(16)]
    cp_dec.wait()
    for i in range(16):
        hv[i] = my[i]

    def dec1(j, c):
        acc = wdv[_O_BE + j]
        for i in range(16):
            acc = acc + hv[i] * wdv[_O_WE + j * 16 + i]
        hv[16 + j] = _atan(acc)
        return c

    lax.fori_loop(0, 16, dec1, jnp.int32(0))

    def dec2(r, c):
        acc = wdv[_O_BD + r]
        for j in range(16):
            acc = acc + hv[16 + j] * wdv[_O_WD + r * 16 + j]
        ov[r] = acc
        return c

    lax.fori_loop(0, 7, dec2, jnp.int32(0))
    ov[7] = zero
    pltpu.sync_copy(ov, out_hbm.at[t])


@jax.jit
def kernel(x, W1, b1, W2, b2, W3, b3, We, be, Wd, bd):
    flat = jnp.concatenate([
        W1.ravel(), b1, W2.ravel(), b2, W3.ravel(), b3,
        jnp.zeros((_W_ROWS - (_O_B3 + 32),), jnp.float32),
    ])
    flatd = jnp.concatenate([
        We.ravel(), be, Wd.ravel(), bd,
        jnp.zeros((_D_ROWS - (_O_BD + 7),), jnp.float32),
    ])
    wsplat = jnp.repeat(flat[:, None], L, axis=1)
    wdsplat = jnp.repeat(flatd[:, None], L, axis=1)
    ps = jnp.repeat(x[:, 0:3].reshape(-1)[:, None], L, axis=1)
    xt = jnp.zeros((NT, 8, L), jnp.float32).at[:, 0:7, :].set(
        x.T.reshape(7, NT, L).transpose(1, 0, 2))

    mesh = plsc.VectorSubcoreMesh(core_axis_name="c", subcore_axis_name="s",
                                  num_cores=1, num_subcores=NT)
    sc = pl.kernel(
        _sc_body,
        out_type=jax.ShapeDtypeStruct((NT, 8, L), jnp.float32),
        mesh=mesh,
        compiler_params=pltpu.CompilerParams(use_tc_tiling_on_sc=False),
        scratch_types=[
            pltpu.VMEM((_W_ROWS, L), jnp.float32),
            pltpu.VMEM((_D_ROWS, L), jnp.float32),
            pltpu.VMEM((3 * N, L), jnp.float32),
            pltpu.VMEM((8, L), jnp.float32),
            pltpu.VMEM((2 * N, L), jnp.float32),
            pltpu.VMEM((64, L), jnp.float32),
            pltpu.VMEM((2 * L, L), jnp.float32),
            pltpu.VMEM((2, L), jnp.float32),
            pltpu.VMEM((2 * NT, L), jnp.float32),
            pltpu.VMEM((NT, L), jnp.float32),
            pltpu.VMEM((1, L), jnp.float32),
            pltpu.VMEM((8, L), jnp.float32),
            pltpu.SemaphoreType.DMA,
            pltpu.SemaphoreType.DMA,
            pltpu.SemaphoreType.DMA,
            pltpu.VMEM_SHARED((NT, L), jnp.float32),
            pltpu.VMEM_SHARED((NT, 2 * N, L), jnp.float32),
            pltpu.VMEM_SHARED((2 * NT, L), jnp.float32),
        ],
    )
    out = sc(wsplat, wdsplat, ps, xt)
    return jnp.transpose(out, (0, 2, 1)).reshape(N, 8)[:, :7]


# speedup vs baseline: 1.2787x; 1.0547x over previous
import numpy as np

import jax
import jax.numpy as jnp
from jax import lax
from jax.experimental import pallas as pl
from jax.experimental.pallas import tpu as pltpu
from jax.experimental.pallas import tpu_sc as plsc

N = 256
DIM_H = 16
CUTOFF = 3.6
L = 16
NT = 16

_O_W1 = 0
_O_B1 = _O_W1 + 16 * 7
_O_W2 = _O_B1 + 16
_O_B2 = _O_W2 + 16 * 16
_O_W3 = _O_B2 + 16
_O_B3 = _O_W3 + 32 * 16
_W_ROWS = ((_O_B3 + 32 + 7) // 8) * 8
_O_WE = 0
_O_BE = _O_WE + 16 * 16
_O_WD = _O_BE + 16
_O_BD = _O_WD + 7 * 16
_D_ROWS = ((_O_BD + 7 + 7) // 8) * 8


def _atan(x):
    t = jnp.abs(x)
    c1 = t > 2.414213562373095
    c2 = t > 0.4142135623730951
    base = jnp.where(c1, np.float32(np.pi / 2),
                     jnp.where(c2, np.float32(np.pi / 4), np.float32(0.0)))
    arg = jnp.where(c1, -1.0 / t, jnp.where(c2, (t - 1.0) / (t + 1.0), t))
    z = arg * arg
    p = (((8.05374449538e-2 * z - 1.38776856032e-1) * z
          + 1.99777106478e-1) * z - 3.33329491539e-1) * z * arg + arg
    return jnp.sign(x) * (base + p)


def _sc_body(w_hbm, wd_hbm, ps_hbm, xt_hbm, out_hbm,
             wv, wdv, posv, xv, tloc, hv, btv, bsv, bbv, av, arow, ov,
             sem1, sem2, sem3, spA, spTT, spB):
    t = lax.axis_index("s")
    f32 = jnp.float32
    iot = lax.broadcasted_iota(jnp.int32, (L,), 0)

    cp_pos = pltpu.async_copy(ps_hbm, posv, sem1)
    cp_dec = pltpu.async_copy(wd_hbm, wdv, sem2)
    cp_w = pltpu.async_copy(w_hbm, wv, sem3)
    pltpu.sync_copy(xt_hbm.at[t], xv)

    zero = jnp.zeros((L,), f32)

    def zstep(k, c):
        for r in range(16):
            tloc[16 * k + r] = zero
        return c

    lax.fori_loop(0, 32, zstep, jnp.int32(0))
    xr = [xv[i] for i in range(7)]
    cp_w.wait()

    def l1(j, c):
        acc = wv[_O_B1 + j]
        for i in range(7):
            acc = acc + xr[i] * wv[_O_W1 + j * 7 + i]
        hv[j] = _atan(acc)
        return c

    lax.fori_loop(0, 16, l1, jnp.int32(0))

    def l2(j, c):
        acc = wv[_O_B2 + j]
        for i in range(16):
            acc = acc + hv[i] * wv[_O_W2 + j * 16 + i]
        hv[16 + j] = _atan(acc)
        return c

    lax.fori_loop(0, 16, l2, jnp.int32(0))

    def l3(j, c):
        acc = wv[_O_B3 + j]
        for i in range(16):
            acc = acc + hv[16 + i] * wv[_O_W3 + j * 16 + i]
        hv[32 + j] = acc
        return c

    lax.fori_loop(0, 32, l3, jnp.int32(0))
    m = [hv[32 + j] for j in range(32)]
    a = m[16] * m[24]
    for j in range(1, 8):
        a = a + m[16 + j] * m[24 + j]

    s = a[0]
    for i in range(1, 16):
        s = jnp.maximum(s, a[i])
    lm = jnp.zeros((L,), f32) + s
    ep = jnp.exp(a - lm)
    arow[0] = a * np.float32(0.0) + lm
    pltpu.sync_copy(arow, spA.at[pl.ds(t, 1)])

    cp_pos.wait()
    big = np.int32(1 << 20)

    def fstep(k4, f):
        for q in range(8):
            kk = 8 * k4 + q
            d = (jnp.abs(posv[3 * kk] - xr[0])
                 + jnp.abs(posv[3 * kk + 1] - xr[1])
                 + jnp.abs(posv[3 * kk + 2] - xr[2]))
            hit = d <= np.float32(CUTOFF)
            f = jnp.minimum(f, jnp.where(hit, kk, big))
        return f

    f = lax.fori_loop(0, N // 8, fstep, jnp.full((L,), 1 << 20, jnp.int32))

    for n in range(16):
        en = ep[n]
        mrow = jnp.where(iot == 0, m[0][n] * en, np.float32(0.0))
        for j in range(1, 16):
            mrow = jnp.where(iot == j, m[j][n] * en, mrow)
        tail = jnp.where(
            iot == 0, en, jnp.where(iot == 1, np.float32(1.0), np.float32(0.0)))
        fn = f[n]
        tloc[2 * fn] = tloc[2 * fn] + mrow
        tloc[2 * fn + 1] = tloc[2 * fn + 1] + tail
    pltpu.sync_copy(tloc, spTT.at[t])

    plsc.subcore_barrier()

    pltpu.sync_copy(spA, av)
    mxv = av[0]
    for c in range(1, NT):
        mxv = jnp.maximum(mxv, av[c])
    amax = jnp.maximum(mxv, np.float32(0.0))
    expneg = jnp.exp(-amax)
    one = np.float32(1.0)
    for r in range(2 * L):
        tloc[r] = jnp.zeros((L,), f32)

    def comb(u, c):
        sc_u = jnp.exp(av[u] - amax)
        sc_odd = jnp.where(iot == 1, one, sc_u)
        pltpu.sync_copy(spTT.at[u, pl.ds(2 * L * t, 2 * L)], btv)
        for r in range(16):
            tloc[2 * r] = tloc[2 * r] + sc_u * btv[2 * r]
            tloc[2 * r + 1] = tloc[2 * r + 1] + sc_odd * btv[2 * r + 1]
        return c

    lax.fori_loop(0, NT, comb, jnp.int32(0))
    cb = [tloc[r] for r in range(2 * L)]
    bs0 = cb[0]
    bs1 = cb[1]
    for r in range(1, 16):
        bs0 = bs0 + cb[2 * r]
        bs1 = bs1 + cb[2 * r + 1]
    bsv[0] = bs0
    bsv[1] = bs1
    pltpu.sync_copy(bsv, spB.at[pl.ds(2 * t, 2)])

    plsc.subcore_barrier()

    pltpu.sync_copy(spB, bbv)
    acc0 = jnp.zeros((L,), f32)
    acc1 = jnp.zeros((L,), f32)
    for u in range(NT):
        g = jnp.where(u < t, np.float32(1.0), np.float32(0.0))
        acc0 = acc0 + g * bbv[2 * u]
        acc1 = acc1 + g * bbv[2 * u + 1]
    srows0 = []
    srows1 = []
    for r in range(16):
        acc0 = acc0 + cb[2 * r]
        acc1 = acc1 + cb[2 * r + 1]
        srows0.append(acc0)
        srows1.append(acc1)

    chans = []
    for j in range(16):
        v = jnp.where(iot == 0, srows0[0][j], np.float32(0.0))
        for r in range(1, 16):
            v = jnp.where(iot == r, srows0[r][j], v)
        chans.append(v)
    pe = jnp.where(iot == 0, srows1[0][0], np.float32(0.0))
    cnt = jnp.where(iot == 0, srows1[0][1], np.float32(0.0))
    for r in range(1, 16):
        pe = jnp.where(iot == r, srows1[r][0], pe)
        cnt = jnp.where(iot == r, srows1[r][1], cnt)

    den = pe + (np.float32(N) - cnt) * expneg
    inv = np.float32(1.0) / den
    my = [chans[j] * inv for j in range(16)]
    cp_dec.wait()
    for i in range(16):
        hv[i] = my[i]

    def dec1(j, c):
        acc = wdv[_O_BE + j]
        for i in range(16):
            acc = acc + hv[i] * wdv[_O_WE + j * 16 + i]
        hv[16 + j] = _atan(acc)
        return c

    lax.fori_loop(0, 16, dec1, jnp.int32(0))

    def dec2(r, c):
        acc = wdv[_O_BD + r]
        for j in range(16):
            acc = acc + hv[16 + j] * wdv[_O_WD + r * 16 + j]
        ov[r] = acc
        return c

    lax.fori_loop(0, 7, dec2, jnp.int32(0))
    ov[7] = zero
    pltpu.sync_copy(ov, out_hbm.at[t])


@jax.jit
def kernel(x, W1, b1, W2, b2, W3, b3, We, be, Wd, bd):
    flat = jnp.concatenate([
        W1.ravel(), b1, W2.ravel(), b2, W3.ravel(), b3,
        jnp.zeros((_W_ROWS - (_O_B3 + 32),), jnp.float32),
    ])
    flatd = jnp.concatenate([
        We.ravel(), be, Wd.ravel(), bd,
        jnp.zeros((_D_ROWS - (_O_BD + 7),), jnp.float32),
    ])
    wsplat = jnp.repeat(flat[:, None], L, axis=1)
    wdsplat = jnp.repeat(flatd[:, None], L, axis=1)
    ps = jnp.repeat(x[:, 0:3].reshape(-1)[:, None], L, axis=1)
    xt = jnp.zeros((NT, 8, L), jnp.float32).at[:, 0:7, :].set(
        x.T.reshape(7, NT, L).transpose(1, 0, 2))

    mesh = plsc.VectorSubcoreMesh(core_axis_name="c", subcore_axis_name="s",
                                  num_cores=1, num_subcores=NT)
    sc = pl.kernel(
        _sc_body,
        out_type=jax.ShapeDtypeStruct((NT, 8, L), jnp.float32),
        mesh=mesh,
        compiler_params=pltpu.CompilerParams(use_tc_tiling_on_sc=False),
        scratch_types=[
            pltpu.VMEM((_W_ROWS, L), jnp.float32),
            pltpu.VMEM((_D_ROWS, L), jnp.float32),
            pltpu.VMEM((3 * N, L), jnp.float32),
            pltpu.VMEM((8, L), jnp.float32),
            pltpu.VMEM((2 * N, L), jnp.float32),
            pltpu.VMEM((64, L), jnp.float32),
            pltpu.VMEM((2 * L, L), jnp.float32),
            pltpu.VMEM((2, L), jnp.float32),
            pltpu.VMEM((2 * NT, L), jnp.float32),
            pltpu.VMEM((NT, L), jnp.float32),
            pltpu.VMEM((1, L), jnp.float32),
            pltpu.VMEM((8, L), jnp.float32),
            pltpu.SemaphoreType.DMA,
            pltpu.SemaphoreType.DMA,
            pltpu.SemaphoreType.DMA,
            pltpu.VMEM_SHARED((NT, L), jnp.float32),
            pltpu.VMEM_SHARED((NT, 2 * N, L), jnp.float32),
            pltpu.VMEM_SHARED((2 * NT, L), jnp.float32),
        ],
    )
    out = sc(wsplat, wdsplat, ps, xt)
    return jnp.transpose(out, (0, 2, 1)).reshape(N, 8)[:, :7]
